# baseline (speedup 1.0000x reference)
import functools

import jax
import jax.numpy as jnp
from jax import lax
from jax.experimental import pallas as pl
from jax.experimental.pallas import tpu as pltpu
from jax.experimental.pallas import tpu_sc as plsc

_ALPHA = 0.7
_K = 205
_N = 1024
_NW = 32
_ROWS = 64 * 768
_RPW = _ROWS // _NW
_GROUPS = _RPW // 16
_NBUCK = 512
_LO = -8.0
_SCALE = _NBUCK / 16.0
_INV = 1.0 / _SCALE
_RANK_TOP = float(_N - _K + 1)
_RANK_BOT = float(_K)


def _sc_body(x_hbm, out_hbm, xb0, xb1, hist, outv, sem0, sem1):
    wid = lax.axis_index("s") * 2 + lax.axis_index("c")
    row0 = wid * _RPW
    lanes = lax.iota(jnp.int32, 16)
    gbase = lanes * _N
    ones = jnp.ones((16,), jnp.float32)
    zeros = jnp.zeros((16,), jnp.float32)
    bufs = (xb0, xb1)
    sems = (sem0, sem1)

    def src(g):
        return x_hbm.at[pl.ds((row0 + g * 16) * _N, 16 * _N)]

    def zinit(i, _):
        for u in range(8):
            hist[pl.ds((i * 8 + u) * 16, 16)] = zeros
        return 0
    lax.fori_loop(0, _NBUCK // 8, zinit, 0)

    pltpu.async_copy(src(0), xb0, sem0)
    pltpu.async_copy(src(1), xb1, sem1)

    def group_body(g, buf, sem):
        pltpu.make_async_copy(src(g), buf, sem).wait()

        def pass_a(i, _):
            for u in range(8):
                v = plsc.load_gather(buf, [gbase + (i * 8 + u)])
                bi = jnp.clip((v - _LO) * _SCALE, 0.0, _NBUCK - 1.0)
                bi = bi.astype(jnp.int32) * 16 + lanes
                plsc.addupdate_scatter(hist, [bi], ones)
            return 0
        lax.fori_loop(0, _N // 8, pass_a, 0)

        def scan(i, carry):
            cum, btop, bbot = carry
            for u in range(8):
                off = (i * 8 + u) * 16
                h = hist[pl.ds(off, 16)]
                hist[pl.ds(off, 16)] = zeros
                cum = cum + h
                btop = btop + jnp.where(cum < _RANK_TOP, 1.0, 0.0)
                bbot = bbot + jnp.where(cum < _RANK_BOT, 1.0, 0.0)
            return cum, btop, bbot
        _, btop, bbot = lax.fori_loop(0, _NBUCK // 8, scan,
                                      (zeros, zeros, zeros))
        t_top = _LO + btop * _INV
        t_bot = _LO + (bbot + 1.0) * _INV

        def pass_b(i, carry):
            s1, s2 = carry
            for u in range(8):
                v = plsc.load_gather(buf, [gbase + (i * 8 + u)])
                s1 = s1 + jnp.maximum(v - t_top, 0.0)
                s2 = s2 + jnp.maximum(t_bot - v, 0.0)
            return s1, s2
        s1, s2 = lax.fori_loop(0, _N // 8, pass_b, (zeros, zeros))

        out = (t_top + s1 * (1.0 / _K)
               + _ALPHA * (t_bot - s2 * (1.0 / _K)))
        outv[pl.ds(g * 16, 16)] = out

    def outer(i, _):
        for b in range(2):
            g = i * 2 + b
            group_body(g, bufs[b], sems[b])

            @pl.when(g + 2 < _GROUPS)
            def _():
                pltpu.async_copy(src(g + 2), bufs[b], sems[b])
        return 0
    lax.fori_loop(0, _GROUPS // 2, outer, 0)

    pltpu.sync_copy(outv, out_hbm.at[pl.ds(row0, _RPW)])


@jax.jit
def _wildcat_sc(flat):
    mesh = plsc.VectorSubcoreMesh(core_axis_name="c", subcore_axis_name="s")
    k = functools.partial(
        pl.kernel,
        mesh=mesh,
        compiler_params=pltpu.CompilerParams(use_tc_tiling_on_sc=False,
                                             needs_layout_passes=False),
        out_type=jax.ShapeDtypeStruct((_ROWS,), jnp.float32),
        scratch_types=[
            pltpu.VMEM((16 * _N,), jnp.float32),
            pltpu.VMEM((16 * _N,), jnp.float32),
            pltpu.VMEM((_NBUCK * 16,), jnp.float32),
            pltpu.VMEM((_RPW,), jnp.float32),
            pltpu.SemaphoreType.DMA,
            pltpu.SemaphoreType.DMA,
        ],
    )(_sc_body)
    return k(flat)


def kernel(input):
    b, c, h, w = input.shape
    flat = input.reshape(b * c * h * w)
    return _wildcat_sc(flat).reshape(b, c)


# speedup vs baseline: 1.6397x; 1.6397x over previous
import functools

import jax
import jax.numpy as jnp
from jax import lax
from jax.experimental import pallas as pl
from jax.experimental.pallas import tpu as pltpu
from jax.experimental.pallas import tpu_sc as plsc

_ALPHA = 0.7
_K = 205
_N = 1024
_NW = 32
_ROWS = 64 * 768
_RPW = _ROWS // _NW
_GROUPS = _RPW // 16
_NBUCK = 512
_LO = -8.0
_SCALE = _NBUCK / 16.0
_INV = 1.0 / _SCALE
_RANK_TOP = float(_N - _K + 1)
_RANK_BOT = float(_K)


def _sc_body(x_hbm, out_hbm, xb0, xb1, hist, outv, sem0, sem1):
    wid = lax.axis_index("s") * 2 + lax.axis_index("c")
    row0 = wid * _RPW
    lanes = lax.iota(jnp.int32, 16)
    gbase = lanes * _N
    ones = jnp.ones((16,), jnp.float32)
    zeros = jnp.zeros((16,), jnp.float32)
    bufs = (xb0, xb1)
    sems = (sem0, sem1)

    def src(g):
        return x_hbm.at[pl.ds((row0 + g * 16) * _N, 16 * _N)]

    @plsc.parallel_loop(0, _NBUCK, unroll=8)
    def _(i):
        hist[pl.ds(i * 16, 16)] = zeros

    pltpu.async_copy(src(0), xb0, sem0)
    pltpu.async_copy(src(1), xb1, sem1)

    def group_body(g, buf, sem):
        pltpu.make_async_copy(src(g), buf, sem).wait()

        @plsc.parallel_loop(0, _N, unroll=8)
        def _(e):
            v = plsc.load_gather(buf, [gbase + e])
            bi = jnp.clip((v - _LO) * _SCALE, 0.0, _NBUCK - 1.0)
            bi = bi.astype(jnp.int32) * 16 + lanes
            plsc.addupdate_scatter(hist, [bi], ones)

        @plsc.parallel_loop(0, _NBUCK, unroll=8,
                            carry=(zeros, zeros, zeros))
        def scan_out(i, carry):
            cum, btop, bbot = carry
            h = hist[pl.ds(i * 16, 16)]
            hist[pl.ds(i * 16, 16)] = zeros
            cum = cum + h
            btop = btop + jnp.where(cum < _RANK_TOP, 1.0, 0.0)
            bbot = bbot + jnp.where(cum < _RANK_BOT, 1.0, 0.0)
            return cum, btop, bbot
        _, btop, bbot = scan_out
        t_top = _LO + btop * _INV
        t_bot = _LO + (bbot + 1.0) * _INV

        @plsc.parallel_loop(0, _N, unroll=8, carry=(zeros, zeros))
        def pass_b(e, carry):
            s1, s2 = carry
            v = plsc.load_gather(buf, [gbase + e])
            s1 = s1 + jnp.maximum(v - t_top, 0.0)
            s2 = s2 + jnp.maximum(t_bot - v, 0.0)
            return s1, s2
        s1, s2 = pass_b

        out = (t_top + s1 * (1.0 / _K)
               + _ALPHA * (t_bot - s2 * (1.0 / _K)))
        outv[pl.ds(g * 16, 16)] = out

    def outer(i, _):
        for b in range(2):
            g = i * 2 + b
            group_body(g, bufs[b], sems[b])

            @pl.when(g + 2 < _GROUPS)
            def _():
                pltpu.async_copy(src(g + 2), bufs[b], sems[b])
        return 0
    lax.fori_loop(0, _GROUPS // 2, outer, 0)

    pltpu.sync_copy(outv, out_hbm.at[pl.ds(row0, _RPW)])


@jax.jit
def _wildcat_sc(flat):
    mesh = plsc.VectorSubcoreMesh(core_axis_name="c", subcore_axis_name="s")
    k = functools.partial(
        pl.kernel,
        mesh=mesh,
        compiler_params=pltpu.CompilerParams(use_tc_tiling_on_sc=False,
                                             needs_layout_passes=False),
        out_type=jax.ShapeDtypeStruct((_ROWS,), jnp.float32),
        scratch_types=[
            pltpu.VMEM((16 * _N,), jnp.float32),
            pltpu.VMEM((16 * _N,), jnp.float32),
            pltpu.VMEM((_NBUCK * 16,), jnp.float32),
            pltpu.VMEM((_RPW,), jnp.float32),
            pltpu.SemaphoreType.DMA,
            pltpu.SemaphoreType.DMA,
        ],
    )(_sc_body)
    return k(flat)


def kernel(input):
    b, c, h, w = input.shape
    flat = input.reshape(b * c * h * w)
    return _wildcat_sc(flat).reshape(b, c)


# speedup vs baseline: 2.6219x; 1.5990x over previous
import functools

import jax
import jax.numpy as jnp
from jax import lax
from jax.experimental import pallas as pl
from jax.experimental.pallas import tpu as pltpu
from jax.experimental.pallas import tpu_sc as plsc

_ALPHA = 0.7
_K = 205
_N = 1024
_NW = 32
_ROWS = 64 * 768
_RPW = _ROWS // _NW
_GROUPS = _RPW // 16
_NBUCK = 512
_PAD = 1040
_LO = -8.0
_SCALE = _NBUCK / 16.0
_INV = 1.0 / _SCALE
_RANK_TOP = float(_N - _K + 1)
_RANK_BOT = float(_K)


def _sc_body(x_hbm, out_hbm, xb0, xb1, hist, outv, sem0, sem1):
    wid = lax.axis_index("s") * 2 + lax.axis_index("c")
    row0 = wid * _RPW
    lanes = lax.iota(jnp.int32, 16)
    gbase = lanes * _PAD
    ones = jnp.ones((16,), jnp.float32)
    zeros = jnp.zeros((16,), jnp.float32)
    bufs = (xb0, xb1)
    sems = (sem0, sem1)

    def dma_group(g, buf, sem, wait):
        for r in range(16):
            cp = pltpu.make_async_copy(
                x_hbm.at[pl.ds((row0 + g * 16 + r) * _N, _N)],
                buf.at[pl.ds(r * _PAD, _N)],
                sem)
            if wait:
                cp.wait()
            else:
                cp.start()

    @plsc.parallel_loop(0, _NBUCK, unroll=8)
    def _(i):
        hist[pl.ds(i * 16, 16)] = zeros

    dma_group(0, xb0, sem0, False)
    dma_group(1, xb1, sem1, False)

    def group_body(g, buf, sem):
        dma_group(g, buf, sem, True)

        @plsc.parallel_loop(0, _N, unroll=8)
        def _(e):
            v = plsc.load_gather(buf, [gbase + e])
            bi = jnp.clip((v - _LO) * _SCALE, 0.0, _NBUCK - 1.0)
            bi = bi.astype(jnp.int32) * 16 + lanes
            plsc.addupdate_scatter(hist, [bi], ones)

        @plsc.parallel_loop(0, _NBUCK, unroll=8,
                            carry=(zeros, zeros, zeros))
        def scan_out(i, carry):
            cum, btop, bbot = carry
            h = hist[pl.ds(i * 16, 16)]
            hist[pl.ds(i * 16, 16)] = zeros
            cum = cum + h
            btop = btop + jnp.where(cum < _RANK_TOP, 1.0, 0.0)
            bbot = bbot + jnp.where(cum < _RANK_BOT, 1.0, 0.0)
            return cum, btop, bbot
        _, btop, bbot = scan_out
        t_top = _LO + btop * _INV
        t_bot = _LO + (bbot + 1.0) * _INV

        @plsc.parallel_loop(0, _N, unroll=8, carry=(zeros, zeros))
        def pass_b(e, carry):
            s1, s2 = carry
            v = plsc.load_gather(buf, [gbase + e])
            s1 = s1 + jnp.maximum(v - t_top, 0.0)
            s2 = s2 + jnp.maximum(t_bot - v, 0.0)
            return s1, s2
        s1, s2 = pass_b

        out = (t_top + s1 * (1.0 / _K)
               + _ALPHA * (t_bot - s2 * (1.0 / _K)))
        outv[pl.ds(g * 16, 16)] = out

    def outer(i, _):
        for b in range(2):
            g = i * 2 + b
            group_body(g, bufs[b], sems[b])

            @pl.when(g + 2 < _GROUPS)
            def _():
                dma_group(g + 2, bufs[b], sems[b], False)
        return 0
    lax.fori_loop(0, _GROUPS // 2, outer, 0)

    pltpu.sync_copy(outv, out_hbm.at[pl.ds(row0, _RPW)])


@jax.jit
def _wildcat_sc(flat):
    mesh = plsc.VectorSubcoreMesh(core_axis_name="c", subcore_axis_name="s")
    k = functools.partial(
        pl.kernel,
        mesh=mesh,
        compiler_params=pltpu.CompilerParams(use_tc_tiling_on_sc=False,
                                             needs_layout_passes=False),
        out_type=jax.ShapeDtypeStruct((_ROWS,), jnp.float32),
        scratch_types=[
            pltpu.VMEM((16 * _PAD,), jnp.float32),
            pltpu.VMEM((16 * _PAD,), jnp.float32),
            pltpu.VMEM((_NBUCK * 16,), jnp.float32),
            pltpu.VMEM((_RPW,), jnp.float32),
            pltpu.SemaphoreType.DMA,
            pltpu.SemaphoreType.DMA,
        ],
    )(_sc_body)
    return k(flat)


def kernel(input):
    b, c, h, w = input.shape
    flat = input.reshape(b * c * h * w)
    return _wildcat_sc(flat).reshape(b, c)
